# Initial kernel scaffold; baseline (speedup 1.0000x reference)
import jax
import jax.numpy as jnp
from jax.experimental import pallas as pl


def kernel(pair_diff, atom_types, sig, eps):
    raise NotImplementedError("write your pallas kernel here")


# speedup vs baseline: 15.4769x; 15.4769x over previous
import functools

import jax
import jax.numpy as jnp
from jax import lax
from jax.experimental import pallas as pl
from jax.experimental.pallas import tpu as pltpu
from jax.experimental.pallas import tpu_sc as plsc

NC = 2
NS = 16
NW = NC * NS
L = 16

_mesh = functools.partial(
    plsc.VectorSubcoreMesh,
    core_axis_name="c",
    subcore_axis_name="s",
    num_cores=NC,
    num_subcores=NS,
)

_params = pltpu.CompilerParams(needs_layout_passes=False)


def _worker_id():
    return lax.axis_index("s") * NC + lax.axis_index("c")


def _d2_body(n_per_w, chunk, pd_hbm, out_hbm, buf, accbuf):
    wid = _worker_id()
    base = wid * n_per_w
    n_chunks = n_per_w // chunk

    def chunk_body(i, acc):
        pltpu.sync_copy(pd_hbm.at[pl.ds(base + i * chunk, chunk)], buf)

        def vec_body(j, a):
            v = buf[pl.ds(j * L, L)]
            return a + v * v

        return lax.fori_loop(0, chunk // L, vec_body, acc)

    acc = lax.fori_loop(0, n_chunks, chunk_body, jnp.zeros((L,), jnp.float32))
    accbuf[...] = acc
    pltpu.sync_copy(accbuf, out_hbm.at[pl.ds(wid * L, L)])


def _lj_body(n_per_w, chunk, n_types, pd_hbm, at_hbm, sig_hbm, eps_hbm,
             part_hbm, en_hbm, fo_hbm,
             sig_v, eps_v, part_v, red_v, pd_v, at_v, en_v, fo_v):
    wid = _worker_id()
    base = wid * n_per_w
    n_chunks = n_per_w // chunk

    pltpu.sync_copy(sig_hbm, sig_v)
    pltpu.sync_copy(eps_hbm, eps_v)
    pltpu.sync_copy(part_hbm, part_v)

    def pr_body(i, a):
        return a + part_v[pl.ds(i * L, L)]

    accv = lax.fori_loop(0, NW, pr_body, jnp.zeros((L,), jnp.float32))

    iota = jnp.arange(L, dtype=jnp.int32)
    for step in (8, 4, 2, 1):
        red_v[...] = accv
        accv = accv + plsc.load_gather(red_v, [jnp.bitwise_xor(iota, step)])
    d2 = accv
    invd2 = 1.0 / d2
    invd6 = invd2 * invd2 * invd2
    c24 = 24.0 * invd2

    def chunk_body(i, _):
        cb = base + i * chunk
        pltpu.sync_copy(pd_hbm.at[pl.ds(cb, chunk)], pd_v)
        pltpu.sync_copy(at_hbm.at[pl.ds(2 * cb, 2 * chunk)], at_v)

        def vec_body(j, __):
            off = j * L
            i2 = (iota + off) * 2
            t0 = plsc.load_gather(at_v, [i2])
            t1 = plsc.load_gather(at_v, [i2 + 1])
            fi = t0 * n_types + t1
            s = plsc.load_gather(sig_v, [fi])
            e = plsc.load_gather(eps_v, [fi])
            pd = pd_v[pl.ds(off, L)]
            r = s / pd
            r2 = r * r
            r4 = r2 * r2
            r6 = r4 * r2
            r12 = r6 * r6
            en = (r12 - r6) * (e * 4.0)
            s2 = s * s
            s4 = s2 * s2
            s6 = s4 * s2
            q6 = s6 * invd6
            q12 = q6 * q6
            fo = (q12 + q12 - q6) * (e * pd) * c24
            en_v[pl.ds(off, L)] = en
            fo_v[pl.ds(off, L)] = fo
            return 0

        lax.fori_loop(0, chunk // L, vec_body, 0)
        pltpu.sync_copy(en_v, en_hbm.at[pl.ds(cb, chunk)])
        pltpu.sync_copy(fo_v, fo_hbm.at[pl.ds(cb, chunk)])
        return 0

    lax.fori_loop(0, n_chunks, chunk_body, 0)


def kernel(pair_diff, atom_types, sig, eps):
    n = pair_diff.shape[0]
    n_types = sig.shape[0]
    assert n % (NW * L) == 0
    n_per_w = n // NW

    chunk1 = 40000
    assert n_per_w % chunk1 == 0 and chunk1 % L == 0

    d2_k = pl.kernel(
        functools.partial(_d2_body, n_per_w, chunk1),
        out_type=jax.ShapeDtypeStruct((NW * L,), jnp.float32),
        mesh=_mesh(),
        compiler_params=_params,
        scratch_types=[
            pltpu.VMEM((chunk1,), jnp.float32),
            pltpu.VMEM((L,), jnp.float32),
        ],
    )
    partials = d2_k(pair_diff)

    chunk2 = 8000
    assert n_per_w % chunk2 == 0 and chunk2 % L == 0

    lj_k = pl.kernel(
        functools.partial(_lj_body, n_per_w, chunk2, n_types),
        out_type=(
            jax.ShapeDtypeStruct((n,), jnp.float32),
            jax.ShapeDtypeStruct((n,), jnp.float32),
        ),
        mesh=_mesh(),
        compiler_params=_params,
        scratch_types=[
            pltpu.VMEM((n_types * n_types,), jnp.float32),
            pltpu.VMEM((n_types * n_types,), jnp.float32),
            pltpu.VMEM((NW * L,), jnp.float32),
            pltpu.VMEM((L,), jnp.float32),
            pltpu.VMEM((chunk2,), jnp.float32),
            pltpu.VMEM((2 * chunk2,), jnp.int32),
            pltpu.VMEM((chunk2,), jnp.float32),
            pltpu.VMEM((chunk2,), jnp.float32),
        ],
    )
    energy, forces = lj_k(
        pair_diff, atom_types.reshape(-1), sig.reshape(-1), eps.reshape(-1),
        partials)
    return (energy, forces)


# speedup vs baseline: 553.9334x; 35.7911x over previous
import functools

import jax
import jax.numpy as jnp
from jax import lax
from jax.experimental import pallas as pl
from jax.experimental.pallas import tpu as pltpu
from jax.experimental.pallas import tpu_sc as plsc

NC = 2
NS = 16
NW = NC * NS
L = 16

_mesh = functools.partial(
    plsc.VectorSubcoreMesh,
    core_axis_name="c",
    subcore_axis_name="s",
    num_cores=NC,
    num_subcores=NS,
)

_params = pltpu.CompilerParams(needs_layout_passes=False)


def _worker_id():
    return lax.axis_index("s") * NC + lax.axis_index("c")


def _d2_body(n_per_w, chunk, pd_hbm, out_hbm, buf, accbuf):
    wid = _worker_id()
    base = wid * n_per_w
    n_chunks = n_per_w // chunk

    def chunk_body(i, acc):
        pltpu.sync_copy(pd_hbm.at[pl.ds(base + i * chunk, chunk)], buf)

        @plsc.parallel_loop(0, chunk, L, unroll=8, carry=acc)
        def vec_body(off, a):
            v = buf[pl.ds(off, L)]
            return a + v * v

        return vec_body

    acc = lax.fori_loop(0, n_chunks, chunk_body, jnp.zeros((L,), jnp.float32))
    accbuf[...] = acc
    pltpu.sync_copy(accbuf, out_hbm.at[pl.ds(wid * L, L)])


def _lj_body(n_per_w, chunk, n_types, pd_hbm, t0_hbm, t1_hbm, sig_hbm,
             eps_hbm, part_hbm, en_hbm, fo_hbm,
             sig_v, eps_v, part_v, red_v, pd_v, t0_v, t1_v, en_v, fo_v):
    wid = _worker_id()
    base = wid * n_per_w
    n_chunks = n_per_w // chunk

    pltpu.sync_copy(sig_hbm, sig_v)
    pltpu.sync_copy(eps_hbm, eps_v)
    pltpu.sync_copy(part_hbm, part_v)

    def pr_body(i, a):
        return a + part_v[pl.ds(i * L, L)]

    accv = lax.fori_loop(0, NW, pr_body, jnp.zeros((L,), jnp.float32))

    iota = jnp.arange(L, dtype=jnp.int32)
    for step in (8, 4, 2, 1):
        red_v[...] = accv
        accv = accv + plsc.load_gather(red_v, [jnp.bitwise_xor(iota, step)])
    d2 = accv
    invd2 = 1.0 / d2
    invd6 = invd2 * invd2 * invd2
    c24 = 24.0 * invd2

    def chunk_body(i, _):
        cb = base + i * chunk
        pltpu.sync_copy(pd_hbm.at[pl.ds(cb, chunk)], pd_v)
        pltpu.sync_copy(t0_hbm.at[pl.ds(cb, chunk)], t0_v)
        pltpu.sync_copy(t1_hbm.at[pl.ds(cb, chunk)], t1_v)

        @plsc.parallel_loop(0, chunk, L, unroll=4)
        def vec_body(off):
            t0 = t0_v[pl.ds(off, L)]
            t1 = t1_v[pl.ds(off, L)]
            fi = t0 * n_types + t1
            s = plsc.load_gather(sig_v, [fi])
            e = plsc.load_gather(eps_v, [fi])
            pd = pd_v[pl.ds(off, L)]
            r = s / pd
            r2 = r * r
            r4 = r2 * r2
            r6 = r4 * r2
            r12 = r6 * r6
            en = (r12 - r6) * (e * 4.0)
            s2 = s * s
            s4 = s2 * s2
            s6 = s4 * s2
            q6 = s6 * invd6
            q12 = q6 * q6
            fo = (q12 + q12 - q6) * (e * pd) * c24
            en_v[pl.ds(off, L)] = en
            fo_v[pl.ds(off, L)] = fo

        pltpu.sync_copy(en_v, en_hbm.at[pl.ds(cb, chunk)])
        pltpu.sync_copy(fo_v, fo_hbm.at[pl.ds(cb, chunk)])
        return 0

    lax.fori_loop(0, n_chunks, chunk_body, 0)


def kernel(pair_diff, atom_types, sig, eps):
    n = pair_diff.shape[0]
    n_types = sig.shape[0]
    assert n % (NW * L) == 0
    n_per_w = n // NW

    chunk1 = 40000
    assert n_per_w % chunk1 == 0 and chunk1 % L == 0

    d2_k = pl.kernel(
        functools.partial(_d2_body, n_per_w, chunk1),
        out_type=jax.ShapeDtypeStruct((NW * L,), jnp.float32),
        mesh=_mesh(),
        compiler_params=_params,
        scratch_types=[
            pltpu.VMEM((chunk1,), jnp.float32),
            pltpu.VMEM((L,), jnp.float32),
        ],
    )
    partials = d2_k(pair_diff)

    chunk2 = 8000
    assert n_per_w % chunk2 == 0 and chunk2 % L == 0

    lj_k = pl.kernel(
        functools.partial(_lj_body, n_per_w, chunk2, n_types),
        out_type=(
            jax.ShapeDtypeStruct((n,), jnp.float32),
            jax.ShapeDtypeStruct((n,), jnp.float32),
        ),
        mesh=_mesh(),
        compiler_params=_params,
        scratch_types=[
            pltpu.VMEM((n_types * n_types,), jnp.float32),
            pltpu.VMEM((n_types * n_types,), jnp.float32),
            pltpu.VMEM((NW * L,), jnp.float32),
            pltpu.VMEM((L,), jnp.float32),
            pltpu.VMEM((chunk2,), jnp.float32),
            pltpu.VMEM((chunk2,), jnp.int32),
            pltpu.VMEM((chunk2,), jnp.int32),
            pltpu.VMEM((chunk2,), jnp.float32),
            pltpu.VMEM((chunk2,), jnp.float32),
        ],
    )
    energy, forces = lj_k(
        pair_diff, atom_types[:, 0], atom_types[:, 1],
        sig.reshape(-1), eps.reshape(-1), partials)
    return (energy, forces)


# speedup vs baseline: 866.5704x; 1.5644x over previous
import functools

import jax
import jax.numpy as jnp
from jax import lax
from jax.experimental import pallas as pl
from jax.experimental.pallas import tpu as pltpu
from jax.experimental.pallas import tpu_sc as plsc

NC = 2
NS = 16
NW = NC * NS
L = 16

_mesh = functools.partial(
    plsc.VectorSubcoreMesh,
    core_axis_name="c",
    subcore_axis_name="s",
    num_cores=NC,
    num_subcores=NS,
)

_params = pltpu.CompilerParams(needs_layout_passes=False)


def _worker_id():
    return lax.axis_index("s") * NC + lax.axis_index("c")


def _d2_body(n_per_w, chunk, pd_hbm, out_hbm, buf, accbuf, sem):
    wid = _worker_id()
    base = wid * n_per_w
    n_chunks = n_per_w // chunk

    def issue(i, b):
        pltpu.async_copy(
            pd_hbm.at[pl.ds(base + i * chunk, chunk)],
            buf.at[pl.ds(b * chunk, chunk)], sem)

    issue(0, 0)

    def chunk_body(i, acc):
        b = jnp.bitwise_and(i, 1)
        pltpu.make_async_copy(
            pd_hbm.at[pl.ds(base + i * chunk, chunk)],
            buf.at[pl.ds(b * chunk, chunk)], sem).wait()

        @pl.when(i + 1 < n_chunks)
        def _():
            issue(i + 1, 1 - b)

        boff = b * chunk

        @plsc.parallel_loop(0, chunk, L, unroll=5, carry=acc)
        def vec_body(off, a):
            v = buf[pl.ds(boff + off, L)]
            return a + v * v

        return vec_body

    acc = lax.fori_loop(0, n_chunks, chunk_body, jnp.zeros((L,), jnp.float32))
    accbuf[...] = acc
    pltpu.sync_copy(accbuf, out_hbm.at[pl.ds(wid * L, L)])


def _lj_body(n_per_w, chunk, n_types, pd_hbm, t0_hbm, t1_hbm, sig_hbm,
             eps_hbm, part_hbm, en_hbm, fo_hbm,
             sig_v, eps_v, part_v, red_v, pd_v, t0_v, t1_v, en_v, fo_v,
             sem_in, sem_out):
    wid = _worker_id()
    base = wid * n_per_w
    n_chunks = n_per_w // chunk

    pltpu.sync_copy(sig_hbm, sig_v)
    pltpu.sync_copy(eps_hbm, eps_v)
    pltpu.sync_copy(part_hbm, part_v)

    def pr_body(i, a):
        return a + part_v[pl.ds(i * L, L)]

    accv = lax.fori_loop(0, NW, pr_body, jnp.zeros((L,), jnp.float32))

    iota = jnp.arange(L, dtype=jnp.int32)
    for step in (8, 4, 2, 1):
        red_v[...] = accv
        accv = accv + plsc.load_gather(red_v, [jnp.bitwise_xor(iota, step)])
    d2 = accv
    invd2 = 1.0 / d2
    invd6 = invd2 * invd2 * invd2
    c24 = 24.0 * invd2

    def issue_in(i, b):
        cb = base + i * chunk
        bo = b * chunk
        pltpu.async_copy(pd_hbm.at[pl.ds(cb, chunk)],
                         pd_v.at[pl.ds(bo, chunk)], sem_in)
        pltpu.async_copy(t0_hbm.at[pl.ds(cb, chunk)],
                         t0_v.at[pl.ds(bo, chunk)], sem_in)
        pltpu.async_copy(t1_hbm.at[pl.ds(cb, chunk)],
                         t1_v.at[pl.ds(bo, chunk)], sem_in)

    def wait_in(i, b):
        cb = base + i * chunk
        bo = b * chunk
        pltpu.make_async_copy(pd_hbm.at[pl.ds(cb, chunk)],
                              pd_v.at[pl.ds(bo, chunk)], sem_in).wait()
        pltpu.make_async_copy(t0_hbm.at[pl.ds(cb, chunk)],
                              t0_v.at[pl.ds(bo, chunk)], sem_in).wait()
        pltpu.make_async_copy(t1_hbm.at[pl.ds(cb, chunk)],
                              t1_v.at[pl.ds(bo, chunk)], sem_in).wait()

    def wait_out(b):
        bo = b * chunk
        pltpu.make_async_copy(en_v.at[pl.ds(bo, chunk)],
                              en_hbm.at[pl.ds(base, chunk)], sem_out).wait()
        pltpu.make_async_copy(fo_v.at[pl.ds(bo, chunk)],
                              fo_hbm.at[pl.ds(base, chunk)], sem_out).wait()

    issue_in(0, 0)

    def chunk_body(i, _):
        b = jnp.bitwise_and(i, 1)
        cb = base + i * chunk
        bo = b * chunk
        wait_in(i, b)

        @pl.when(i + 1 < n_chunks)
        def _():
            issue_in(i + 1, 1 - b)

        @pl.when(i >= 2)
        def _():
            wait_out(b)

        @plsc.parallel_loop(0, chunk, L, unroll=4)
        def vec_body(off):
            t0 = t0_v[pl.ds(bo + off, L)]
            t1 = t1_v[pl.ds(bo + off, L)]
            fi = t0 * n_types + t1
            s = plsc.load_gather(sig_v, [fi])
            e = plsc.load_gather(eps_v, [fi])
            pd = pd_v[pl.ds(bo + off, L)]
            r = s / pd
            r2 = r * r
            r4 = r2 * r2
            r6 = r4 * r2
            r12 = r6 * r6
            en = (r12 - r6) * (e * 4.0)
            s2 = s * s
            s4 = s2 * s2
            s6 = s4 * s2
            q6 = s6 * invd6
            q12 = q6 * q6
            fo = (q12 + q12 - q6) * (e * pd) * c24
            en_v[pl.ds(bo + off, L)] = en
            fo_v[pl.ds(bo + off, L)] = fo

        pltpu.async_copy(en_v.at[pl.ds(bo, chunk)],
                         en_hbm.at[pl.ds(cb, chunk)], sem_out)
        pltpu.async_copy(fo_v.at[pl.ds(bo, chunk)],
                         fo_hbm.at[pl.ds(cb, chunk)], sem_out)
        return 0

    lax.fori_loop(0, n_chunks, chunk_body, 0)
    wait_out(jnp.int32(0))
    wait_out(jnp.int32(1))


def kernel(pair_diff, atom_types, sig, eps):
    n = pair_diff.shape[0]
    n_types = sig.shape[0]
    assert n % (NW * L) == 0
    n_per_w = n // NW

    chunk1 = 20000
    assert n_per_w % chunk1 == 0 and chunk1 % L == 0

    d2_k = pl.kernel(
        functools.partial(_d2_body, n_per_w, chunk1),
        out_type=jax.ShapeDtypeStruct((NW * L,), jnp.float32),
        mesh=_mesh(),
        compiler_params=_params,
        scratch_types=[
            pltpu.VMEM((2 * chunk1,), jnp.float32),
            pltpu.VMEM((L,), jnp.float32),
            pltpu.SemaphoreType.DMA,
        ],
    )
    partials = d2_k(pair_diff)

    chunk2 = 8000
    assert n_per_w % chunk2 == 0 and chunk2 % L == 0

    lj_k = pl.kernel(
        functools.partial(_lj_body, n_per_w, chunk2, n_types),
        out_type=(
            jax.ShapeDtypeStruct((n,), jnp.float32),
            jax.ShapeDtypeStruct((n,), jnp.float32),
        ),
        mesh=_mesh(),
        compiler_params=_params,
        scratch_types=[
            pltpu.VMEM((n_types * n_types,), jnp.float32),
            pltpu.VMEM((n_types * n_types,), jnp.float32),
            pltpu.VMEM((NW * L,), jnp.float32),
            pltpu.VMEM((L,), jnp.float32),
            pltpu.VMEM((2 * chunk2,), jnp.float32),
            pltpu.VMEM((2 * chunk2,), jnp.int32),
            pltpu.VMEM((2 * chunk2,), jnp.int32),
            pltpu.VMEM((2 * chunk2,), jnp.float32),
            pltpu.VMEM((2 * chunk2,), jnp.float32),
            pltpu.SemaphoreType.DMA,
            pltpu.SemaphoreType.DMA,
        ],
    )
    energy, forces = lj_k(
        pair_diff, atom_types[:, 0], atom_types[:, 1],
        sig.reshape(-1), eps.reshape(-1), partials)
    return (energy, forces)
